# baseline (speedup 1.0000x reference)
import functools

import jax
import jax.numpy as jnp
from jax import lax
from jax.experimental import pallas as pl
from jax.experimental.pallas import tpu as pltpu
from jax.experimental.pallas import tpu_sc as plsc

VOCAB = 1000000
DIM = 64
BATCH = 4096
HIST = 200
B = BATCH * HIST

NC = 2
NS = 16
NW = NC * NS
B_PER_W = B // NW
CHUNK = 1024
NCHUNK = B_PER_W // CHUNK

_mesh = plsc.VectorSubcoreMesh(core_axis_name="c", subcore_axis_name="s")


@functools.partial(
    pl.kernel,
    mesh=_mesh,
    out_type=jax.ShapeDtypeStruct((B, DIM), jnp.float32),
    scratch_types=[
        pltpu.VMEM((CHUNK,), jnp.int32),
        pltpu.VMEM((CHUNK, DIM), jnp.float32),
        pltpu.SemaphoreType.DMA,
    ],
    compiler_params=pltpu.CompilerParams(use_tc_tiling_on_sc=False),
)
def _gather(table_hbm, idx_hbm, out_hbm, idx_v, rows_v, sem):
    wid = lax.axis_index("s") * NC + lax.axis_index("c")
    base = wid * B_PER_W

    def body(i, carry):
        off = base + i * CHUNK
        pltpu.sync_copy(idx_hbm.at[pl.ds(off, CHUNK)], idx_v)
        pltpu.async_copy(table_hbm.at[idx_v], rows_v, sem).wait()
        pltpu.sync_copy(rows_v, out_hbm.at[pl.ds(off, CHUNK)])
        return carry

    lax.fori_loop(0, NCHUNK, body, 0)


def kernel(inputs, table):
    idx = inputs.reshape(B)
    out = _gather(table, idx)
    return out.reshape(BATCH, HIST, DIM)


# speedup vs baseline: 1.0129x; 1.0129x over previous
import functools

import jax
import jax.numpy as jnp
from jax import lax
from jax.experimental import pallas as pl
from jax.experimental.pallas import tpu as pltpu
from jax.experimental.pallas import tpu_sc as plsc

VOCAB = 1000000
DIM = 64
BATCH = 4096
HIST = 200
B = BATCH * HIST

NC = 2
NS = 16
NW = NC * NS
B_PER_W = B // NW
CHUNK = 800
NCHUNK = B_PER_W // CHUNK
NBUF = 2
NPAIR = NCHUNK // NBUF

_mesh = plsc.VectorSubcoreMesh(core_axis_name="c", subcore_axis_name="s")


@functools.partial(
    pl.kernel,
    mesh=_mesh,
    out_type=jax.ShapeDtypeStruct((B, DIM), jnp.float32),
    scratch_types=[
        pltpu.VMEM((NBUF, CHUNK), jnp.int32),
        pltpu.VMEM((NBUF, CHUNK, DIM), jnp.float32),
        pltpu.SemaphoreType.DMA,
        pltpu.SemaphoreType.DMA,
        pltpu.SemaphoreType.DMA,
    ],
    compiler_params=pltpu.CompilerParams(use_tc_tiling_on_sc=False),
)
def _gather(table_hbm, idx_hbm, out_hbm, idx_v, rows_v, sem_l, sem_g, sem_w):
    wid = lax.axis_index("s") * NC + lax.axis_index("c")
    base = wid * B_PER_W

    def load(i, b):
        pltpu.async_copy(
            idx_hbm.at[pl.ds(base + i * CHUNK, CHUNK)], idx_v.at[b], sem_l)

    def wait_load(b):
        pltpu.make_async_copy(
            idx_hbm.at[pl.ds(base, CHUNK)], idx_v.at[b], sem_l).wait()

    def gather(b):
        return pltpu.async_copy(table_hbm.at[idx_v.at[b]], rows_v.at[b], sem_g)

    def write(i, b):
        pltpu.async_copy(
            rows_v.at[b], out_hbm.at[pl.ds(base + i * CHUNK, CHUNK)], sem_w)

    def wait_write(b):
        pltpu.make_async_copy(
            rows_v.at[b], out_hbm.at[pl.ds(base, CHUNK)], sem_w).wait()

    for b in range(NBUF):
        load(b, b)
    for b in range(NBUF):
        wait_load(b)
        gather(b).wait()
        write(b, b)
        load(b + NBUF, b)

    def body(k, carry):
        for b in range(NBUF):
            i = k * NBUF + b
            wait_write(b)
            wait_load(b)
            gather(b).wait()
            write(i, b)
            load(i + NBUF, b)
        return carry

    lax.fori_loop(1, NPAIR - 1, body, 0)

    for b in range(NBUF):
        i = NCHUNK - NBUF + b
        wait_write(b)
        wait_load(b)
        gather(b).wait()
        write(i, b)
    for b in range(NBUF):
        wait_write(b)


def kernel(inputs, table):
    idx = inputs.reshape(B)
    out = _gather(table, idx)
    return out.reshape(BATCH, HIST, DIM)
